# Initial kernel scaffold; baseline (speedup 1.0000x reference)
import jax
import jax.numpy as jnp
from jax.experimental import pallas as pl


def kernel(x, edge_index, edge_attr, W1, b1, W2, b2, W_root, b_conv, W_lin, b_lin):
    raise NotImplementedError("write your pallas kernel here")


# speedup vs baseline: 3.7944x; 3.7944x over previous
import functools

import jax
import jax.numpy as jnp
from jax import lax
from jax.experimental import pallas as pl
from jax.experimental.pallas import tpu as pltpu
from jax.experimental.pallas import tpu_sc as plsc

NC = 2
NS = 16
NW = NC * NS
CH = 128


def _gather_kernel(V, D, B):
    nchunk = pl.cdiv(B, CH)
    per_tile = pl.cdiv(nchunk, NW)
    mesh = plsc.VectorSubcoreMesh(core_axis_name="c", subcore_axis_name="s")

    @functools.partial(
        pl.kernel,
        mesh=mesh,
        out_type=jax.ShapeDtypeStruct((B, D), jnp.float32),
        scratch_types=[
            pltpu.VMEM((CH,), jnp.int32),
            pltpu.VMEM((CH, D), jnp.float32),
            pltpu.SemaphoreType.DMA,
        ],
    )
    def gk(idx_hbm, table_hbm, out_hbm, idx_v, rows_v, sem):
        wid = lax.axis_index("s") * NC + lax.axis_index("c")

        def body(j, carry):
            c = wid + j * NW

            @pl.when(c < nchunk)
            def _():
                off = pl.multiple_of(c * CH, CH)
                pltpu.sync_copy(idx_hbm.at[pl.ds(off, CH)], idx_v)
                pltpu.async_copy(table_hbm.at[idx_v], rows_v, sem).wait()
                pltpu.sync_copy(rows_v, out_hbm.at[pl.ds(off, CH)])

            return carry

        lax.fori_loop(0, per_tile, body, 0)

    return gk


def _scatter_kernel(V, B, D):
    nchunk = pl.cdiv(B, CH)
    per_tile = pl.cdiv(nchunk, NW)
    rows_per_sub = V // NS
    mesh = plsc.VectorSubcoreMesh(core_axis_name="c", subcore_axis_name="s")

    @functools.partial(
        pl.kernel,
        mesh=mesh,
        out_type=jax.ShapeDtypeStruct((NC, V, D), jnp.float32),
        scratch_types=[
            pltpu.VMEM((CH,), jnp.int32),
            pltpu.VMEM((CH, D), jnp.float32),
            pltpu.VMEM_SHARED((V, D), jnp.float32),
        ],
    )
    def sk(dst_hbm, msgs_hbm, zeros_hbm, out_hbm, idx_v, rows_v, acc_sh):
        cid = lax.axis_index("c")
        sid = lax.axis_index("s")
        wid = sid * NC + cid

        row0 = pl.multiple_of(sid * rows_per_sub, 8)
        pltpu.sync_copy(zeros_hbm.at[pl.ds(row0, rows_per_sub)],
                        acc_sh.at[pl.ds(row0, rows_per_sub)])
        plsc.subcore_barrier()

        def body(j, carry):
            c = wid + j * NW

            @pl.when(c < nchunk)
            def _():
                off = pl.multiple_of(c * CH, CH)
                pltpu.sync_copy(dst_hbm.at[pl.ds(off, CH)], idx_v)
                pltpu.sync_copy(msgs_hbm.at[pl.ds(off, CH)], rows_v)
                pltpu.sync_copy(rows_v, acc_sh.at[idx_v], add=True)

            return carry

        lax.fori_loop(0, per_tile, body, 0)
        plsc.subcore_barrier()
        pltpu.sync_copy(acc_sh.at[pl.ds(row0, rows_per_sub)],
                        out_hbm.at[cid, pl.ds(row0, rows_per_sub)])

    return sk


def _msgs_body(ea_ref, xj_ref, w1_ref, b1_ref, w2m_ref, r_ref, s_ref, out_ref):
    h = jnp.maximum(
        jnp.dot(ea_ref[...], w1_ref[...], preferred_element_type=jnp.float32)
        + b1_ref[...], 0.0)
    hx = jnp.dot(h, r_ref[...], preferred_element_type=jnp.float32)
    z = jnp.dot(xj_ref[...], w2m_ref[...], preferred_element_type=jnp.float32)
    m = jnp.dot(z * hx, s_ref[...], preferred_element_type=jnp.float32)
    lane = lax.broadcasted_iota(jnp.int32, m.shape, 1)
    out_ref[...] = m + (lane == 8).astype(jnp.float32)


def _final_body(x_ref, acc_ref, wr_ref, bc_ref, wl_ref, bl_ref, out_ref):
    a = acc_ref[0] + acc_ref[1]
    agg = a[:, :8] / jnp.maximum(a[:, 8:9], 1.0)
    c = (jnp.dot(x_ref[...], wr_ref[...], preferred_element_type=jnp.float32)
         + agg + bc_ref[...])
    out_ref[...] = (jnp.dot(jnp.maximum(c, 0.0), wl_ref[...],
                            preferred_element_type=jnp.float32) + bl_ref[...])


def kernel(x, edge_index, edge_attr, W1, b1, W2, b2, W_root, b_conv, W_lin,
           b_lin):
    V, IN_DIM = x.shape
    E, EDGE_DIM = edge_attr.shape
    HID = W_root.shape[1]
    OUT = W_lin.shape[1]
    KA = W2.shape[0] + 1
    KP = 40
    ZW = KA * HID

    src = edge_index[0].astype(jnp.int32)
    dst = edge_index[1].astype(jnp.int32)

    f32 = jnp.float32
    W1p = jnp.zeros((EDGE_DIM, KP), f32).at[:, :KA - 1].set(W1)
    b1p = jnp.zeros((1, KP), f32).at[0, :KA - 1].set(b1).at[0, KA - 1].set(1.0)
    W2a = jnp.concatenate([W2, b2[None, :]], axis=0)
    W2m = W2a.reshape(KA, IN_DIM, HID).transpose(1, 0, 2).reshape(IN_DIM, ZW)
    kcol = jnp.arange(ZW) // HID
    R = (jnp.arange(KP)[:, None] == kcol[None, :]).astype(f32)
    MW = 128
    S = ((jnp.arange(ZW)[:, None] % HID)
         == jnp.arange(MW)[None, :]).astype(f32)

    xj = _gather_kernel(V, IN_DIM, E)(src, x)

    TE = 2000
    msgs16 = pl.pallas_call(
        _msgs_body,
        grid=(E // TE,),
        in_specs=[
            pl.BlockSpec((TE, EDGE_DIM), lambda i: (i, 0)),
            pl.BlockSpec((TE, IN_DIM), lambda i: (i, 0)),
            pl.BlockSpec((EDGE_DIM, KP), lambda i: (0, 0)),
            pl.BlockSpec((1, KP), lambda i: (0, 0)),
            pl.BlockSpec((IN_DIM, ZW), lambda i: (0, 0)),
            pl.BlockSpec((KP, ZW), lambda i: (0, 0)),
            pl.BlockSpec((ZW, MW), lambda i: (0, 0)),
        ],
        out_specs=pl.BlockSpec((TE, MW), lambda i: (i, 0)),
        out_shape=jax.ShapeDtypeStruct((E, MW), f32),
    )(edge_attr, xj, W1p, b1p, W2m, R, S)

    V_pad = ((V + 8 * NS - 1) // (8 * NS)) * (8 * NS)
    zeros_acc = jnp.zeros((V_pad, MW), f32)
    acc = _scatter_kernel(V_pad, E, MW)(dst, msgs16, zeros_acc)

    TN = 2000
    out = pl.pallas_call(
        _final_body,
        grid=(V // TN,),
        in_specs=[
            pl.BlockSpec((TN, IN_DIM), lambda i: (i, 0)),
            pl.BlockSpec((NC, TN, MW), lambda i: (0, i, 0)),
            pl.BlockSpec((IN_DIM, HID), lambda i: (0, 0)),
            pl.BlockSpec((1, HID), lambda i: (0, 0)),
            pl.BlockSpec((HID, OUT), lambda i: (0, 0)),
            pl.BlockSpec((1, OUT), lambda i: (0, 0)),
        ],
        out_specs=pl.BlockSpec((TN, OUT), lambda i: (i, 0)),
        out_shape=jax.ShapeDtypeStruct((V, OUT), f32),
    )(x, acc, W_root, b_conv.reshape(1, HID), W_lin, b_lin.reshape(1, OUT))
    return out


# speedup vs baseline: 4.0114x; 1.0572x over previous
import functools

import jax
import jax.numpy as jnp
from jax import lax
from jax.experimental import pallas as pl
from jax.experimental.pallas import tpu as pltpu
from jax.experimental.pallas import tpu_sc as plsc

NC = 2
NS = 16
NW = NC * NS
CH = 128


def _gather_kernel(V, D, B):
    nchunk = pl.cdiv(B, CH)
    per_tile = pl.cdiv(nchunk, NW)
    mesh = plsc.VectorSubcoreMesh(core_axis_name="c", subcore_axis_name="s")

    @functools.partial(
        pl.kernel,
        mesh=mesh,
        out_type=jax.ShapeDtypeStruct((B, D), jnp.float32),
        scratch_types=[
            pltpu.VMEM((CH,), jnp.int32),
            pltpu.VMEM((CH,), jnp.int32),
            pltpu.VMEM((CH, D), jnp.float32),
            pltpu.VMEM((CH, D), jnp.float32),
            pltpu.SemaphoreType.DMA,
            pltpu.SemaphoreType.DMA,
        ],
    )
    def gk(idx_hbm, table_hbm, out_hbm, idx0, idx1, rows0, rows1, sem0, sem1):
        wid = lax.axis_index("s") * NC + lax.axis_index("c")
        bufs = ((idx0, rows0, sem0), (idx1, rows1, sem1))

        def start(j, b):
            c = wid + j * NW

            @pl.when(c < nchunk)
            def _():
                idx_v, rows_v, sem = bufs[b]
                off = pl.multiple_of(c * CH, CH)
                pltpu.sync_copy(idx_hbm.at[pl.ds(off, CH)], idx_v)
                pltpu.async_copy(table_hbm.at[idx_v], rows_v, sem)

        def drain(j, b):
            c = wid + j * NW

            @pl.when(c < nchunk)
            def _():
                idx_v, rows_v, sem = bufs[b]
                off = pl.multiple_of(c * CH, CH)
                pltpu.make_async_copy(table_hbm.at[idx_v], rows_v, sem).wait()
                pltpu.sync_copy(rows_v, out_hbm.at[pl.ds(off, CH)])

        start(0, 0)

        def body(j, carry):
            @pl.when(j % 2 == 0)
            def _():
                start(j + 1, 1)
                drain(j, 0)

            @pl.when(j % 2 == 1)
            def _():
                start(j + 1, 0)
                drain(j, 1)

            return carry

        lax.fori_loop(0, per_tile - 1, body, 0)
        last = per_tile - 1
        drain(last, last % 2)

    return gk


def _scatter_kernel(V, B, D):
    nchunk = pl.cdiv(B, CH)
    per_tile = pl.cdiv(nchunk, NW)
    rows_per_sub = V // NS
    mesh = plsc.VectorSubcoreMesh(core_axis_name="c", subcore_axis_name="s")

    @functools.partial(
        pl.kernel,
        mesh=mesh,
        out_type=jax.ShapeDtypeStruct((NC, V, D), jnp.float32),
        scratch_types=[
            pltpu.VMEM((CH,), jnp.int32),
            pltpu.VMEM((CH, D), jnp.float32),
            pltpu.VMEM_SHARED((V, D), jnp.float32),
        ],
    )
    def sk(dst_hbm, msgs_hbm, zeros_hbm, out_hbm, idx_v, rows_v, acc_sh):
        cid = lax.axis_index("c")
        sid = lax.axis_index("s")
        wid = sid * NC + cid

        row0 = pl.multiple_of(sid * rows_per_sub, 8)
        pltpu.sync_copy(zeros_hbm.at[pl.ds(row0, rows_per_sub)],
                        acc_sh.at[pl.ds(row0, rows_per_sub)])
        plsc.subcore_barrier()

        def body(j, carry):
            c = wid + j * NW

            @pl.when(c < nchunk)
            def _():
                off = pl.multiple_of(c * CH, CH)
                pltpu.sync_copy(dst_hbm.at[pl.ds(off, CH)], idx_v)
                pltpu.sync_copy(msgs_hbm.at[pl.ds(off, CH)], rows_v)
                pltpu.sync_copy(rows_v, acc_sh.at[idx_v], add=True)

            return carry

        lax.fori_loop(0, per_tile, body, 0)
        plsc.subcore_barrier()
        pltpu.sync_copy(acc_sh.at[pl.ds(row0, rows_per_sub)],
                        out_hbm.at[cid, pl.ds(row0, rows_per_sub)])

    return sk


def _msgs_body(ea_ref, xj_ref, w1_ref, b1_ref, w2m_ref, r_ref, s_ref, out_ref):
    h = jnp.maximum(
        jnp.dot(ea_ref[...], w1_ref[...], preferred_element_type=jnp.float32)
        + b1_ref[...], 0.0)
    hx = jnp.dot(h, r_ref[...], preferred_element_type=jnp.float32)
    z = jnp.dot(xj_ref[...], w2m_ref[...], preferred_element_type=jnp.float32)
    m = jnp.dot(z * hx, s_ref[...], preferred_element_type=jnp.float32)
    lane = lax.broadcasted_iota(jnp.int32, m.shape, 1)
    out_ref[...] = m + (lane == 8).astype(jnp.float32)


def _final_body(x_ref, acc_ref, wr_ref, bc_ref, wl_ref, bl_ref, out_ref):
    a = acc_ref[0] + acc_ref[1]
    agg = a[:, :8] / jnp.maximum(a[:, 8:9], 1.0)
    c = (jnp.dot(x_ref[...], wr_ref[...], preferred_element_type=jnp.float32)
         + agg + bc_ref[...])
    out_ref[...] = (jnp.dot(jnp.maximum(c, 0.0), wl_ref[...],
                            preferred_element_type=jnp.float32) + bl_ref[...])


def kernel(x, edge_index, edge_attr, W1, b1, W2, b2, W_root, b_conv, W_lin,
           b_lin):
    V, IN_DIM = x.shape
    E, EDGE_DIM = edge_attr.shape
    HID = W_root.shape[1]
    OUT = W_lin.shape[1]
    KA = W2.shape[0] + 1
    KP = 40
    ZW = KA * HID

    src = edge_index[0].astype(jnp.int32)
    dst = edge_index[1].astype(jnp.int32)

    f32 = jnp.float32
    W1p = jnp.zeros((EDGE_DIM, KP), f32).at[:, :KA - 1].set(W1)
    b1p = jnp.zeros((1, KP), f32).at[0, :KA - 1].set(b1).at[0, KA - 1].set(1.0)
    W2a = jnp.concatenate([W2, b2[None, :]], axis=0)
    W2m = W2a.reshape(KA, IN_DIM, HID).transpose(1, 0, 2).reshape(IN_DIM, ZW)
    kcol = jnp.arange(ZW) // HID
    R = (jnp.arange(KP)[:, None] == kcol[None, :]).astype(f32)
    MW = 128
    S = ((jnp.arange(ZW)[:, None] % HID)
         == jnp.arange(MW)[None, :]).astype(f32)

    xj = _gather_kernel(V, IN_DIM, E)(src, x)

    TE = 2000
    msgs16 = pl.pallas_call(
        _msgs_body,
        grid=(E // TE,),
        in_specs=[
            pl.BlockSpec((TE, EDGE_DIM), lambda i: (i, 0)),
            pl.BlockSpec((TE, IN_DIM), lambda i: (i, 0)),
            pl.BlockSpec((EDGE_DIM, KP), lambda i: (0, 0)),
            pl.BlockSpec((1, KP), lambda i: (0, 0)),
            pl.BlockSpec((IN_DIM, ZW), lambda i: (0, 0)),
            pl.BlockSpec((KP, ZW), lambda i: (0, 0)),
            pl.BlockSpec((ZW, MW), lambda i: (0, 0)),
        ],
        out_specs=pl.BlockSpec((TE, MW), lambda i: (i, 0)),
        out_shape=jax.ShapeDtypeStruct((E, MW), f32),
    )(edge_attr, xj, W1p, b1p, W2m, R, S)

    V_pad = ((V + 8 * NS - 1) // (8 * NS)) * (8 * NS)
    zeros_acc = jnp.zeros((V_pad, MW), f32)
    acc = _scatter_kernel(V_pad, E, MW)(dst, msgs16, zeros_acc)

    TN = 2000
    out = pl.pallas_call(
        _final_body,
        grid=(V // TN,),
        in_specs=[
            pl.BlockSpec((TN, IN_DIM), lambda i: (i, 0)),
            pl.BlockSpec((NC, TN, MW), lambda i: (0, i, 0)),
            pl.BlockSpec((IN_DIM, HID), lambda i: (0, 0)),
            pl.BlockSpec((1, HID), lambda i: (0, 0)),
            pl.BlockSpec((HID, OUT), lambda i: (0, 0)),
            pl.BlockSpec((1, OUT), lambda i: (0, 0)),
        ],
        out_specs=pl.BlockSpec((TN, OUT), lambda i: (i, 0)),
        out_shape=jax.ShapeDtypeStruct((V, OUT), f32),
    )(x, acc, W_root, b_conv.reshape(1, HID), W_lin, b_lin.reshape(1, OUT))
    return out


# speedup vs baseline: 4.6022x; 1.1473x over previous
import functools

import jax
import jax.numpy as jnp
from jax import lax
from jax.experimental import pallas as pl
from jax.experimental.pallas import tpu as pltpu
from jax.experimental.pallas import tpu_sc as plsc

NC = 2
NS = 16
NW = NC * NS
CH = 128


def _gather_kernel(V, D, B):
    nchunk = pl.cdiv(B, CH)
    per_tile = pl.cdiv(nchunk, NW)
    mesh = plsc.VectorSubcoreMesh(core_axis_name="c", subcore_axis_name="s")

    @functools.partial(
        pl.kernel,
        mesh=mesh,
        out_type=jax.ShapeDtypeStruct((B, D), jnp.float32),
        scratch_types=[
            pltpu.VMEM((CH,), jnp.int32),
            pltpu.VMEM((CH,), jnp.int32),
            pltpu.VMEM((CH, D), jnp.float32),
            pltpu.VMEM((CH, D), jnp.float32),
            pltpu.SemaphoreType.DMA,
            pltpu.SemaphoreType.DMA,
        ],
    )
    def gk(idx_hbm, table_hbm, out_hbm, idx0, idx1, rows0, rows1, sem0, sem1):
        wid = lax.axis_index("s") * NC + lax.axis_index("c")
        bufs = ((idx0, rows0, sem0), (idx1, rows1, sem1))

        def start(j, b):
            c = wid + j * NW

            @pl.when(c < nchunk)
            def _():
                idx_v, rows_v, sem = bufs[b]
                off = pl.multiple_of(c * CH, CH)
                pltpu.sync_copy(idx_hbm.at[pl.ds(off, CH)], idx_v)
                pltpu.async_copy(table_hbm.at[idx_v], rows_v, sem)

        def drain(j, b):
            c = wid + j * NW

            @pl.when(c < nchunk)
            def _():
                idx_v, rows_v, sem = bufs[b]
                off = pl.multiple_of(c * CH, CH)
                pltpu.make_async_copy(table_hbm.at[idx_v], rows_v, sem).wait()
                pltpu.sync_copy(rows_v, out_hbm.at[pl.ds(off, CH)])

        start(0, 0)

        def body(j, carry):
            @pl.when(j % 2 == 0)
            def _():
                start(j + 1, 1)
                drain(j, 0)

            @pl.when(j % 2 == 1)
            def _():
                start(j + 1, 0)
                drain(j, 1)

            return carry

        lax.fori_loop(0, per_tile - 1, body, 0)
        last = per_tile - 1
        drain(last, last % 2)

    return gk


def _scatter_kernel(V, B, D):
    nchunk = pl.cdiv(B, CH)
    per_tile = pl.cdiv(nchunk, NW)
    rows_per_sub = V // NS
    mesh = plsc.VectorSubcoreMesh(core_axis_name="c", subcore_axis_name="s")

    @functools.partial(
        pl.kernel,
        mesh=mesh,
        out_type=jax.ShapeDtypeStruct((NC, V, D), jnp.float32),
        scratch_types=[
            pltpu.VMEM((CH,), jnp.int32),
            pltpu.VMEM((CH,), jnp.int32),
            pltpu.VMEM((CH, D), jnp.float32),
            pltpu.VMEM((CH, D), jnp.float32),
            pltpu.SemaphoreType.DMA,
            pltpu.SemaphoreType.DMA,
            pltpu.VMEM_SHARED((V, D), jnp.float32),
        ],
    )
    def sk(dst_hbm, msgs_hbm, zeros_hbm, out_hbm, idx0, idx1, rows0, rows1,
           sem0, sem1, acc_sh):
        cid = lax.axis_index("c")
        sid = lax.axis_index("s")
        wid = sid * NC + cid
        bufs = ((idx0, rows0, sem0), (idx1, rows1, sem1))

        row0 = pl.multiple_of(sid * rows_per_sub, 8)
        pltpu.sync_copy(zeros_hbm.at[pl.ds(row0, rows_per_sub)],
                        acc_sh.at[pl.ds(row0, rows_per_sub)])
        plsc.subcore_barrier()

        def start(j, b):
            c = wid + j * NW

            @pl.when(c < nchunk)
            def _():
                idx_v, rows_v, sem = bufs[b]
                off = pl.multiple_of(c * CH, CH)
                pltpu.async_copy(dst_hbm.at[pl.ds(off, CH)], idx_v, sem)
                pltpu.async_copy(msgs_hbm.at[pl.ds(off, CH)], rows_v, sem)

        def process(j, b):
            c = wid + j * NW

            @pl.when(c < nchunk)
            def _():
                idx_v, rows_v, sem = bufs[b]
                off = pl.multiple_of(c * CH, CH)
                pltpu.make_async_copy(dst_hbm.at[pl.ds(off, CH)], idx_v,
                                      sem).wait()
                pltpu.make_async_copy(msgs_hbm.at[pl.ds(off, CH)], rows_v,
                                      sem).wait()
                pltpu.sync_copy(rows_v, acc_sh.at[idx_v], add=True)

        start(0, 0)

        def body(j, carry):
            @pl.when(j % 2 == 0)
            def _():
                start(j + 1, 1)
                process(j, 0)

            @pl.when(j % 2 == 1)
            def _():
                start(j + 1, 0)
                process(j, 1)

            return carry

        lax.fori_loop(0, per_tile - 1, body, 0)
        last = per_tile - 1
        process(last, last % 2)
        plsc.subcore_barrier()
        pltpu.sync_copy(acc_sh.at[pl.ds(row0, rows_per_sub)],
                        out_hbm.at[cid, pl.ds(row0, rows_per_sub)])

    return sk


def _msgs_body(ea_ref, xj_ref, w1_ref, b1_ref, w2m_ref, r_ref, s_ref, out_ref):
    h = jnp.maximum(
        jnp.dot(ea_ref[...], w1_ref[...], preferred_element_type=jnp.float32)
        + b1_ref[...], 0.0)
    hx = jnp.dot(h, r_ref[...], preferred_element_type=jnp.float32)
    z = jnp.dot(xj_ref[...], w2m_ref[...], preferred_element_type=jnp.float32)
    m = jnp.dot(z * hx, s_ref[...], preferred_element_type=jnp.float32)
    lane = lax.broadcasted_iota(jnp.int32, m.shape, 1)
    out_ref[...] = m + (lane == 8).astype(jnp.float32)


def _final_body(x_ref, acc_ref, wr_ref, bc_ref, wl_ref, bl_ref, out_ref):
    a = acc_ref[0] + acc_ref[1]
    agg = a[:, :8] / jnp.maximum(a[:, 8:9], 1.0)
    c = (jnp.dot(x_ref[...], wr_ref[...], preferred_element_type=jnp.float32)
         + agg + bc_ref[...])
    out_ref[...] = (jnp.dot(jnp.maximum(c, 0.0), wl_ref[...],
                            preferred_element_type=jnp.float32) + bl_ref[...])


def kernel(x, edge_index, edge_attr, W1, b1, W2, b2, W_root, b_conv, W_lin,
           b_lin):
    V, IN_DIM = x.shape
    E, EDGE_DIM = edge_attr.shape
    HID = W_root.shape[1]
    OUT = W_lin.shape[1]
    KA = W2.shape[0] + 1
    KP = 40
    ZW = KA * HID

    src = edge_index[0].astype(jnp.int32)
    dst = edge_index[1].astype(jnp.int32)

    f32 = jnp.float32
    W1p = jnp.zeros((EDGE_DIM, KP), f32).at[:, :KA - 1].set(W1)
    b1p = jnp.zeros((1, KP), f32).at[0, :KA - 1].set(b1).at[0, KA - 1].set(1.0)
    W2a = jnp.concatenate([W2, b2[None, :]], axis=0)
    W2m = W2a.reshape(KA, IN_DIM, HID).transpose(1, 0, 2).reshape(IN_DIM, ZW)
    kcol = jnp.arange(ZW) // HID
    R = (jnp.arange(KP)[:, None] == kcol[None, :]).astype(f32)
    MW = 128
    S = ((jnp.arange(ZW)[:, None] % HID)
         == jnp.arange(MW)[None, :]).astype(f32)

    xj = _gather_kernel(V, IN_DIM, E)(src, x)

    TE = 2000
    msgs16 = pl.pallas_call(
        _msgs_body,
        grid=(E // TE,),
        in_specs=[
            pl.BlockSpec((TE, EDGE_DIM), lambda i: (i, 0)),
            pl.BlockSpec((TE, IN_DIM), lambda i: (i, 0)),
            pl.BlockSpec((EDGE_DIM, KP), lambda i: (0, 0)),
            pl.BlockSpec((1, KP), lambda i: (0, 0)),
            pl.BlockSpec((IN_DIM, ZW), lambda i: (0, 0)),
            pl.BlockSpec((KP, ZW), lambda i: (0, 0)),
            pl.BlockSpec((ZW, MW), lambda i: (0, 0)),
        ],
        out_specs=pl.BlockSpec((TE, MW), lambda i: (i, 0)),
        out_shape=jax.ShapeDtypeStruct((E, MW), f32),
    )(edge_attr, xj, W1p, b1p, W2m, R, S)

    V_pad = ((V + 8 * NS - 1) // (8 * NS)) * (8 * NS)
    zeros_acc = jnp.zeros((V_pad, MW), f32)
    acc = _scatter_kernel(V_pad, E, MW)(dst, msgs16, zeros_acc)

    TN = 2000
    out = pl.pallas_call(
        _final_body,
        grid=(V // TN,),
        in_specs=[
            pl.BlockSpec((TN, IN_DIM), lambda i: (i, 0)),
            pl.BlockSpec((NC, TN, MW), lambda i: (0, i, 0)),
            pl.BlockSpec((IN_DIM, HID), lambda i: (0, 0)),
            pl.BlockSpec((1, HID), lambda i: (0, 0)),
            pl.BlockSpec((HID, OUT), lambda i: (0, 0)),
            pl.BlockSpec((1, OUT), lambda i: (0, 0)),
        ],
        out_specs=pl.BlockSpec((TN, OUT), lambda i: (i, 0)),
        out_shape=jax.ShapeDtypeStruct((V, OUT), f32),
    )(x, acc, W_root, b_conv.reshape(1, HID), W_lin, b_lin.reshape(1, OUT))
    return out
